# baseline (speedup 1.0000x reference)
import functools

import jax
import jax.numpy as jnp
from jax import lax
from jax.experimental import pallas as pl
from jax.experimental.pallas import tpu as pltpu, tpu_sc as plsc

DIM = 64
B = 16384
GAMMA = 12.0

NUM_CORES = 2
NUM_SUBCORES = 16
NUM_WORKERS = NUM_CORES * NUM_SUBCORES
B_PER_W = B // NUM_WORKERS
CHUNK = 128
NCHUNK = B_PER_W // CHUNK
LANES = 16
NGROUP = B_PER_W // LANES


def _sc_body(heads_hbm, rels_hbm, tails_hbm, ent_hbm, rel_hbm, out_hbm,
             idx_h, idx_r, idx_t, hr_v, t_v, out_v, sem):
    wid = lax.axis_index("s") * NUM_CORES + lax.axis_index("c")

    pltpu.sync_copy(heads_hbm.at[wid], idx_h)
    pltpu.sync_copy(rels_hbm.at[wid], idx_r)
    pltpu.sync_copy(tails_hbm.at[wid], idx_t)

    for j in range(NCHUNK):
        pltpu.async_copy(ent_hbm.at[idx_h.at[j]],
                         hr_v.at[pl.ds(j * CHUNK, CHUNK)], sem)
        pltpu.async_copy(ent_hbm.at[idx_t.at[j]],
                         t_v.at[pl.ds(j * CHUNK, CHUNK)], sem)
    for j in range(2 * NCHUNK):
        pltpu.make_async_copy(ent_hbm.at[idx_h.at[0]],
                              hr_v.at[pl.ds(0, CHUNK)], sem).wait()
    for j in range(NCHUNK):
        pltpu.async_copy(rel_hbm.at[idx_r.at[j]],
                         hr_v.at[pl.ds(j * CHUNK, CHUNK)], sem, add=True)
    for j in range(NCHUNK):
        pltpu.make_async_copy(rel_hbm.at[idx_r.at[0]],
                              hr_v.at[pl.ds(0, CHUNK)], sem).wait()

    lane_iota = lax.iota(jnp.int32, LANES)

    def group_body(g, carry):
        rows = g * LANES + lane_iota
        acc = jnp.zeros((LANES,), jnp.float32)
        for d in range(DIM):
            col = jnp.full((LANES,), d, jnp.int32)
            hr = plsc.load_gather(hr_v, [rows, col])
            t = plsc.load_gather(t_v, [rows, col])
            acc = acc + jnp.abs(hr - t)
        out_v[pl.ds(g * LANES, LANES)] = GAMMA - acc
        return carry

    lax.fori_loop(0, NGROUP, group_body, 0)

    pltpu.sync_copy(out_v, out_hbm.at[pl.ds(wid * B_PER_W, B_PER_W)])


@jax.jit
def _score(heads, rels, tails, entity_embedding, relation_embedding):
    mesh = plsc.VectorSubcoreMesh(
        core_axis_name="c", subcore_axis_name="s",
        num_cores=NUM_CORES, num_subcores=NUM_SUBCORES)
    kern = functools.partial(
        pl.kernel, mesh=mesh,
        out_type=jax.ShapeDtypeStruct((B,), jnp.float32),
        scratch_types=[
            pltpu.VMEM((NCHUNK, CHUNK), jnp.int32),
            pltpu.VMEM((NCHUNK, CHUNK), jnp.int32),
            pltpu.VMEM((NCHUNK, CHUNK), jnp.int32),
            pltpu.VMEM((B_PER_W, DIM), jnp.float32),
            pltpu.VMEM((B_PER_W, DIM), jnp.float32),
            pltpu.VMEM((B_PER_W,), jnp.float32),
            pltpu.SemaphoreType.DMA,
        ],
        compiler_params=pltpu.CompilerParams(
            needs_layout_passes=False, use_tc_tiling_on_sc=False),
    )(_sc_body)
    return kern(heads, rels, tails, entity_embedding, relation_embedding)


def kernel(sample, entity_embedding, relation_embedding):
    idx = sample.astype(jnp.int32).T.reshape(3, NUM_WORKERS, NCHUNK, CHUNK)
    score = _score(idx[0], idx[1], idx[2], entity_embedding,
                   relation_embedding)
    return score.reshape(B, 1)


# speedup vs baseline: 1.0084x; 1.0084x over previous
import functools

import jax
import jax.numpy as jnp
from jax import lax
from jax.experimental import pallas as pl
from jax.experimental.pallas import tpu as pltpu, tpu_sc as plsc

DIM = 64
B = 16384
GAMMA = 12.0

NUM_CORES = 2
NUM_SUBCORES = 16
NUM_WORKERS = NUM_CORES * NUM_SUBCORES
B_PER_W = B // NUM_WORKERS
CHUNK = 128
NCHUNK = B_PER_W // CHUNK
LANES = 16
NGROUP = CHUNK // LANES


def _sc_body(ent2, rel2, hrow, rrow, trow, hpar, rpar, tpar, out_hbm,
             ih, ir, it, ph, pr, pt, hrows, rrows, trows, out_v, sem):
    wid = lax.axis_index("s") * NUM_CORES + lax.axis_index("c")
    base = wid * B_PER_W
    lane_iota = lax.iota(jnp.int32, LANES)

    def chunk_body(ch, carry):
        cbase = base + ch * CHUNK
        pltpu.sync_copy(hrow.at[pl.ds(cbase, CHUNK)], ih)
        pltpu.sync_copy(rrow.at[pl.ds(cbase, CHUNK)], ir)
        pltpu.sync_copy(trow.at[pl.ds(cbase, CHUNK)], it)
        pltpu.sync_copy(hpar.at[pl.ds(cbase, CHUNK)], ph)
        pltpu.sync_copy(rpar.at[pl.ds(cbase, CHUNK)], pr)
        pltpu.sync_copy(tpar.at[pl.ds(cbase, CHUNK)], pt)
        pltpu.async_copy(ent2.at[ih], hrows, sem)
        pltpu.async_copy(rel2.at[ir], rrows, sem)
        pltpu.async_copy(ent2.at[it], trows, sem)
        pltpu.make_async_copy(ent2.at[ih], hrows, sem).wait()
        pltpu.make_async_copy(rel2.at[ir], rrows, sem).wait()
        pltpu.make_async_copy(ent2.at[it], trows, sem).wait()

        def grp_body(g, carry):
            hp = ph[pl.ds(g * LANES, LANES)]
            rp = pr[pl.ds(g * LANES, LANES)]
            tp = pt[pl.ds(g * LANES, LANES)]

            def smp_body(l, vacc):
                s = g * LANES + l
                lsplat = jnp.full((LANES,), l, jnp.int32)
                hsel = jnp.take_along_axis(hp, lsplat, axis=0) != 0
                rsel = jnp.take_along_axis(rp, lsplat, axis=0) != 0
                tsel = jnp.take_along_axis(tp, lsplat, axis=0) != 0
                acc = jnp.zeros((LANES,), jnp.float32)
                for k in range(DIM // LANES):
                    hv = jnp.where(hsel, hrows[s, pl.ds(64 + k * 16, 16)],
                                   hrows[s, pl.ds(k * 16, 16)])
                    rv = jnp.where(rsel, rrows[s, pl.ds(64 + k * 16, 16)],
                                   rrows[s, pl.ds(k * 16, 16)])
                    tv = jnp.where(tsel, trows[s, pl.ds(64 + k * 16, 16)],
                                   trows[s, pl.ds(k * 16, 16)])
                    acc = acc + jnp.abs(hv + rv - tv)
                tot = jnp.sum(acc)
                return jnp.where(lane_iota == l, tot, vacc)

            vacc = lax.fori_loop(0, LANES, smp_body,
                                 jnp.zeros((LANES,), jnp.float32))
            out_v[pl.ds(ch * CHUNK + g * LANES, LANES)] = GAMMA - vacc
            return carry

        lax.fori_loop(0, NGROUP, grp_body, 0)
        return carry

    lax.fori_loop(0, NCHUNK, chunk_body, 0)
    pltpu.sync_copy(out_v, out_hbm.at[pl.ds(base, B_PER_W)])


@jax.jit
def _score(ent2, rel2, hrow, rrow, trow, hpar, rpar, tpar):
    mesh = plsc.VectorSubcoreMesh(
        core_axis_name="c", subcore_axis_name="s",
        num_cores=NUM_CORES, num_subcores=NUM_SUBCORES)
    kern = functools.partial(
        pl.kernel, mesh=mesh,
        out_type=jax.ShapeDtypeStruct((B,), jnp.float32),
        scratch_types=[
            pltpu.VMEM((CHUNK,), jnp.int32),
            pltpu.VMEM((CHUNK,), jnp.int32),
            pltpu.VMEM((CHUNK,), jnp.int32),
            pltpu.VMEM((CHUNK,), jnp.int32),
            pltpu.VMEM((CHUNK,), jnp.int32),
            pltpu.VMEM((CHUNK,), jnp.int32),
            pltpu.VMEM((CHUNK, 128), jnp.float32),
            pltpu.VMEM((CHUNK, 128), jnp.float32),
            pltpu.VMEM((CHUNK, 128), jnp.float32),
            pltpu.VMEM((B_PER_W,), jnp.float32),
            pltpu.SemaphoreType.DMA,
        ],
        compiler_params=pltpu.CompilerParams(
            needs_layout_passes=False, use_tc_tiling_on_sc=True),
    )(_sc_body)
    return kern(ent2, rel2, hrow, rrow, trow, hpar, rpar, tpar)


def kernel(sample, entity_embedding, relation_embedding):
    idx = sample.astype(jnp.int32)
    score = _score(
        entity_embedding.reshape(500000, 128),
        relation_embedding.reshape(500000, 128),
        idx[:, 0] >> 1, idx[:, 1] >> 1, idx[:, 2] >> 1,
        idx[:, 0] & 1, idx[:, 1] & 1, idx[:, 2] & 1)
    return score.reshape(B, 1)
